# baseline (speedup 1.0000x reference)
import jax
import jax.numpy as jnp
from jax import lax
from jax.experimental import pallas as pl
from jax.experimental.pallas import tpu as pltpu
from jax.experimental.pallas import tpu_sc as plsc

B = 16384
D = 4096
K = 128
NC, NS = 2, 16
NW = NC * NS
ROWS_PER_W = B // NW


def _body(x_hbm, idx_hbm, out_hbm, idx_v, vals_v, sem_cp, sem_idx, sem_sc):
    wid = lax.axis_index("s") * NC + lax.axis_index("c")
    row0 = wid * ROWS_PER_W

    copy = pltpu.make_async_copy(
        x_hbm.at[pl.ds(row0 * D, ROWS_PER_W * D)],
        out_hbm.at[pl.ds(row0 * D, ROWS_PER_W * D)],
        sem_cp,
    )
    copy.start()

    pltpu.make_async_copy(
        idx_hbm.at[pl.ds(row0, ROWS_PER_W)], idx_v, sem_idx
    ).start()

    half = jnp.full((16,), 0.5, dtype=jnp.float32)
    for j in range(K // 16):
        vals_v[pl.ds(j * 16, 16)] = half

    pltpu.make_async_copy(
        idx_hbm.at[pl.ds(row0, ROWS_PER_W)], idx_v, sem_idx
    ).wait()
    copy.wait()

    def fire(r, carry):
        pltpu.make_async_copy(vals_v, out_hbm.at[idx_v.at[r]], sem_sc).start()
        return carry

    lax.fori_loop(0, ROWS_PER_W, fire, None)

    def drain(r, carry):
        pltpu.make_async_copy(vals_v, out_hbm.at[idx_v.at[r]], sem_sc).wait()
        return carry

    lax.fori_loop(0, ROWS_PER_W, drain, None)


_mesh = plsc.VectorSubcoreMesh(
    core_axis_name="c", subcore_axis_name="s", num_cores=NC, num_subcores=NS)

_scatter_call = pl.kernel(
    _body,
    out_type=jax.ShapeDtypeStruct((B * D,), jnp.float32),
    mesh=_mesh,
    compiler_params=pltpu.CompilerParams(needs_layout_passes=False),
    scratch_types=[
        pltpu.VMEM((ROWS_PER_W, K), jnp.int32),
        pltpu.VMEM((K,), jnp.float32),
        pltpu.SemaphoreType.DMA,
        pltpu.SemaphoreType.DMA,
        pltpu.SemaphoreType.DMA,
    ],
)


def kernel(x, index):
    rows = jnp.arange(B, dtype=jnp.int32)[:, None]
    flat_idx = rows * D + index.astype(jnp.int32)
    flat = _scatter_call(x.reshape(B * D), flat_idx)
    return flat.reshape(B, D)


# speedup vs baseline: 2.9055x; 2.9055x over previous
import jax
import jax.numpy as jnp
from jax import lax
from jax.experimental import pallas as pl
from jax.experimental.pallas import tpu as pltpu
from jax.experimental.pallas import tpu_sc as plsc

B = 16384
D = 4096
K = 128
NC, NS = 2, 16
NW = NC * NS
ROWS_PER_W = B // NW
TCR = 512


def _copy_body(x_ref, o_ref):
    o_ref[...] = x_ref[...]


_tc_copy = pl.pallas_call(
    _copy_body,
    grid=(B // TCR,),
    in_specs=[pl.BlockSpec((TCR, D), lambda i: (i, 0))],
    out_specs=pl.BlockSpec((TCR, D), lambda i: (i, 0)),
    out_shape=jax.ShapeDtypeStruct((B, D), jnp.float32),
)


def _sc_body(out_hbm, idx_hbm, idx_v, vals_v, sem_idx, sem_sc):
    wid = lax.axis_index("s") * NC + lax.axis_index("c")
    row0 = wid * ROWS_PER_W

    pltpu.make_async_copy(
        idx_hbm.at[pl.ds(row0, ROWS_PER_W)], idx_v, sem_idx
    ).start()

    half = jnp.full((16,), 0.5, dtype=jnp.float32)
    for j in range(K // 16):
        vals_v[pl.ds(j * 16, 16)] = half

    pltpu.make_async_copy(
        idx_hbm.at[pl.ds(row0, ROWS_PER_W)], idx_v, sem_idx
    ).wait()

    def fire(r, carry):
        pltpu.make_async_copy(vals_v, out_hbm.at[idx_v.at[r]], sem_sc).start()
        return carry

    lax.fori_loop(0, ROWS_PER_W, fire, None)

    def drain(r, carry):
        pltpu.make_async_copy(vals_v, out_hbm.at[idx_v.at[r]], sem_sc).wait()
        return carry

    lax.fori_loop(0, ROWS_PER_W, drain, None)


_mesh = plsc.VectorSubcoreMesh(
    core_axis_name="c", subcore_axis_name="s", num_cores=NC, num_subcores=NS)

_sc_scatter = pl.kernel(
    _sc_body,
    mesh=_mesh,
    compiler_params=pltpu.CompilerParams(needs_layout_passes=False),
    scratch_types=[
        pltpu.VMEM((ROWS_PER_W, K), jnp.int32),
        pltpu.VMEM((K,), jnp.float32),
        pltpu.SemaphoreType.DMA,
        pltpu.SemaphoreType.DMA,
    ],
)


def kernel(x, index):
    rows = jnp.arange(B, dtype=jnp.int32)[:, None]
    flat_idx = rows * D + index.astype(jnp.int32)
    copied = _tc_copy(x)
    buf = jax.new_ref(copied.reshape(B * D))
    _sc_scatter(buf, flat_idx)
    return buf[...].reshape(B, D)
